# baseline (speedup 1.0000x reference)
import jax
import jax.numpy as jnp
from jax.experimental import pallas as pl
from jax.experimental.pallas import tpu as pltpu

_DN_NT = (((0,), (0,)), ((), ()))
_DN_NN = (((1,), (0,)), ((), ()))

_PI_HI = 3.140625
_PI_LO = 3.1415926535897931 - 3.140625
_INV_PI = 0.3183098861837907
_SIN_C = (0.9999999827737748, -0.16666651514235015, 0.008332963909001756,
          -0.00019804748134769412, 2.5980951125369577e-06)
_COS_C = (0.9999999998456133, -0.4999999951142117, 0.04166664187638778,
          -0.001388843233082876, 2.47637666162959e-05,
          -2.611494973412389e-07)


def _sincos(kp):
    q = jnp.round(kp * _INV_PI)
    r = (kp - q * _PI_HI) - q * _PI_LO
    parity = jnp.bitwise_and(q.astype(jnp.int32), 1).astype(jnp.float32)
    sign = 1.0 - 2.0 * parity
    r2 = r * r
    s = _SIN_C[4]
    for k in (3, 2, 1, 0):
        s = s * r2 + _SIN_C[k]
    s = s * r
    c = _COS_C[5]
    for k in (4, 3, 2, 1, 0):
        c = c * r2 + _COS_C[k]
    return sign * s, sign * c


def _split_f32(a):
    hi = a.astype(jnp.bfloat16).astype(jnp.float32)
    return hi, a - hi


def _dot3_f32(a, b, dn):
    ah, al = _split_f32(a)
    bh, bl = _split_f32(b)

    def d(x, y):
        return jax.lax.dot_general(x, y, dn,
                                   preferred_element_type=jnp.float32)

    return d(ah, bh) + d(ah, bl) + d(al, bh)


def _split_b16(a):
    hi = a.astype(jnp.bfloat16)
    return hi, (a - hi.astype(jnp.float32)).astype(jnp.bfloat16)


def _dot3_b16(ah, al, bh, bl, dn):
    def d(x, y):
        return jax.lax.dot_general(x, y, dn,
                                   preferred_element_type=jnp.float32)

    return d(ah, bh) + d(ah, bl) + d(al, bh)


def _lri_kernel(kv_ref, kvx_ref, kvy_ref, kvz_ref, pos_ref, batch_ref, h_ref,
                w1_ref, b1_ref, w2_ref, b2_ref, w3_ref, b3_ref, out_ref):
    pos = pos_ref[...]
    batch = batch_ref[...]
    h = h_ref[...]
    n_k = kvx_ref.shape[1]
    bk = 8 * n_k

    x = _dot3_f32(kv_ref[...], w1_ref[...], _DN_NN) + b1_ref[...]
    x = jax.nn.gelu(x)
    x = _dot3_f32(x, w2_ref[...], _DN_NN) + b2_ref[...]
    x = jax.nn.gelu(x)
    filt = _dot3_f32(x, w3_ref[...], _DN_NN) + b3_ref[...]

    seg_cols = jax.lax.broadcasted_iota(jnp.int32, (1, 8), 1)
    oh16 = (batch == seg_cols).astype(jnp.bfloat16)

    def gather8(tbl_ref):
        t_hi, t_lo = _split_b16(tbl_ref[...])
        return (jax.lax.dot_general(oh16, t_hi, _DN_NN,
                                    preferred_element_type=jnp.float32)
                + jax.lax.dot_general(oh16, t_lo, _DN_NN,
                                      preferred_element_type=jnp.float32))

    kp = (pos[:, 0:1] * gather8(kvx_ref)
          + pos[:, 1:2] * gather8(kvy_ref)
          + pos[:, 2:3] * gather8(kvz_ref))

    sin_kp, cos_kp = _sincos(kp)
    c_hi, c_lo = _split_b16(cos_kp)
    s_hi, s_lo = _split_b16(sin_kp)

    cols = jax.lax.broadcasted_iota(jnp.int32, (1, bk), 1) // n_k
    mask = (batch == cols).astype(jnp.bfloat16)

    def tile(a):
        return jnp.concatenate([a] * 8, axis=1)

    mc_hi = tile(c_hi) * mask
    mc_lo = tile(c_lo) * mask
    ms_hi = tile(s_hi) * mask
    ms_lo = tile(s_lo) * mask

    h_hi, h_lo = _split_b16(h)
    s_re = _dot3_b16(mc_hi, mc_lo, h_hi, h_lo, _DN_NT)
    s_im = -_dot3_b16(ms_hi, ms_lo, h_hi, h_lo, _DN_NT)

    t_re = s_re * filt
    t_im = s_im * filt
    tr_hi, tr_lo = _split_b16(t_re)
    ti_hi, ti_lo = _split_b16(t_im)
    out_ref[...] = (_dot3_b16(mc_hi, mc_lo, tr_hi, tr_lo, _DN_NN)
                    - _dot3_b16(ms_hi, ms_lo, ti_hi, ti_lo, _DN_NN))


def kernel(k_vectors, positions, batch, h, W1, b1, W2, b2, W3, b3):
    B, N_K, _ = k_vectors.shape
    N, D = h.shape
    kv = k_vectors.reshape(B * N_K, 3)
    kvx = k_vectors[:, :, 0]
    kvy = k_vectors[:, :, 1]
    kvz = k_vectors[:, :, 2]
    batch2 = batch.astype(jnp.int32).reshape(N, 1)
    return pl.pallas_call(
        _lri_kernel,
        out_shape=jax.ShapeDtypeStruct((N, D), jnp.float32),
        compiler_params=pltpu.CompilerParams(
            vmem_limit_bytes=112 * 1024 * 1024),
    )(kv, kvx, kvy, kvz, positions, batch2, h,
      W1, b1.reshape(1, D), W2, b2.reshape(1, D), W3, b3.reshape(1, D))


# speedup vs baseline: 1.3269x; 1.3269x over previous
import jax
import jax.numpy as jnp
from jax.experimental import pallas as pl
from jax.experimental.pallas import tpu as pltpu

_DN_NT = (((0,), (0,)), ((), ()))
_DN_NN = (((1,), (0,)), ((), ()))

_PI_HI = 3.140625
_PI_LO = 3.1415926535897931 - 3.140625
_INV_PI = 0.3183098861837907
_SIN_C = (0.9999999827737748, -0.16666651514235015, 0.008332963909001756,
          -0.00019804748134769412, 2.5980951125369577e-06)
_COS_C = (0.9999999998456133, -0.4999999951142117, 0.04166664187638778,
          -0.001388843233082876, 2.47637666162959e-05,
          -2.611494973412389e-07)


def _sincos(kp):
    q = jnp.round(kp * _INV_PI)
    r = (kp - q * _PI_HI) - q * _PI_LO
    parity = jnp.bitwise_and(q.astype(jnp.int32), 1).astype(jnp.float32)
    sign = 1.0 - 2.0 * parity
    r2 = r * r
    s = _SIN_C[4]
    for k in (3, 2, 1, 0):
        s = s * r2 + _SIN_C[k]
    s = s * r
    c = _COS_C[5]
    for k in (4, 3, 2, 1, 0):
        c = c * r2 + _COS_C[k]
    return sign * s, sign * c


def _split_f32(a):
    hi = a.astype(jnp.bfloat16).astype(jnp.float32)
    return hi, a - hi


def _dot3_f32(a, b, dn):
    ah, al = _split_f32(a)
    bh, bl = _split_f32(b)

    def d(x, y):
        return jax.lax.dot_general(x, y, dn,
                                   preferred_element_type=jnp.float32)

    return d(ah, bh) + d(ah, bl) + d(al, bh)


def _split_b16(a):
    hi = a.astype(jnp.bfloat16)
    return hi, (a - hi.astype(jnp.float32)).astype(jnp.bfloat16)


def _dot3_b16(ah, al, bh, bl, dn):
    def d(x, y):
        return jax.lax.dot_general(x, y, dn,
                                   preferred_element_type=jnp.float32)

    return d(ah, bh)


def _lri_kernel(kv_ref, kvx_ref, kvy_ref, kvz_ref, pos_ref, batch_ref, h_ref,
                w1_ref, b1_ref, w2_ref, b2_ref, w3_ref, b3_ref, out_ref):
    pos = pos_ref[...]
    batch = batch_ref[...]
    h = h_ref[...]
    n_k = kvx_ref.shape[1]
    bk = 8 * n_k

    x = _dot3_f32(kv_ref[...], w1_ref[...], _DN_NN) + b1_ref[...]
    x = jax.nn.gelu(x)
    x = _dot3_f32(x, w2_ref[...], _DN_NN) + b2_ref[...]
    x = jax.nn.gelu(x)
    filt = _dot3_f32(x, w3_ref[...], _DN_NN) + b3_ref[...]

    seg_cols = jax.lax.broadcasted_iota(jnp.int32, (1, 8), 1)
    oh16 = (batch == seg_cols).astype(jnp.bfloat16)

    def gather8(tbl_ref):
        t_hi, t_lo = _split_b16(tbl_ref[...])
        return (jax.lax.dot_general(oh16, t_hi, _DN_NN,
                                    preferred_element_type=jnp.float32)
                + jax.lax.dot_general(oh16, t_lo, _DN_NN,
                                      preferred_element_type=jnp.float32))

    kp = (pos[:, 0:1] * gather8(kvx_ref)
          + pos[:, 1:2] * gather8(kvy_ref)
          + pos[:, 2:3] * gather8(kvz_ref))

    sin_kp, cos_kp = _sincos(kp)
    c_hi, c_lo = _split_b16(cos_kp)
    s_hi, s_lo = _split_b16(sin_kp)

    cols = jax.lax.broadcasted_iota(jnp.int32, (1, bk), 1) // n_k
    mask = (batch == cols).astype(jnp.bfloat16)

    def tile(a):
        return jnp.concatenate([a] * 8, axis=1)

    mc_hi = tile(c_hi) * mask
    mc_lo = tile(c_lo) * mask
    ms_hi = tile(s_hi) * mask
    ms_lo = tile(s_lo) * mask

    h_hi, h_lo = _split_b16(h)
    s_re = _dot3_b16(mc_hi, mc_lo, h_hi, h_lo, _DN_NT)
    s_im = -_dot3_b16(ms_hi, ms_lo, h_hi, h_lo, _DN_NT)

    t_re = s_re * filt
    t_im = s_im * filt
    tr_hi, tr_lo = _split_b16(t_re)
    ti_hi, ti_lo = _split_b16(t_im)
    out_ref[...] = (_dot3_b16(mc_hi, mc_lo, tr_hi, tr_lo, _DN_NN)
                    - _dot3_b16(ms_hi, ms_lo, ti_hi, ti_lo, _DN_NN))


def kernel(k_vectors, positions, batch, h, W1, b1, W2, b2, W3, b3):
    B, N_K, _ = k_vectors.shape
    N, D = h.shape
    kv = k_vectors.reshape(B * N_K, 3)
    kvx = k_vectors[:, :, 0]
    kvy = k_vectors[:, :, 1]
    kvz = k_vectors[:, :, 2]
    batch2 = batch.astype(jnp.int32).reshape(N, 1)
    return pl.pallas_call(
        _lri_kernel,
        out_shape=jax.ShapeDtypeStruct((N, D), jnp.float32),
        compiler_params=pltpu.CompilerParams(
            vmem_limit_bytes=112 * 1024 * 1024),
    )(kv, kvx, kvy, kvz, positions, batch2, h,
      W1, b1.reshape(1, D), W2, b2.reshape(1, D), W3, b3.reshape(1, D))
